# baseline (speedup 1.0000x reference)
import functools

import jax
import jax.numpy as jnp
from jax import lax
from jax.experimental import pallas as pl
from jax.experimental.pallas import tpu as pltpu
from jax.experimental.pallas import tpu_sc as plsc

N = 10000
E = 320000
NC, NS = 2, 16
NW = NC * NS
CH = 128
K = 80
EPW = K * CH
E_PAD = NW * EPW
NB = 10240
NPAD = NB + 8
STRIPE = NB // NS
ZR = 32

_mesh = plsc.VectorSubcoreMesh(core_axis_name="c", subcore_axis_name="s")


@functools.partial(
    pl.kernel,
    out_type=(jax.ShapeDtypeStruct((NC, NB, 16), jnp.float32),
              jax.ShapeDtypeStruct((NC, NB, 16), jnp.float32)),
    mesh=_mesh,
    scratch_types=[
        pltpu.VMEM_SHARED((NPAD, 16), jnp.float32),
        pltpu.VMEM_SHARED((NPAD, 16), jnp.float32),
        pltpu.VMEM((K, CH), jnp.int32),
        pltpu.VMEM((K, CH), jnp.int32),
        pltpu.VMEM((CH, 16), jnp.float32),
        pltpu.VMEM((STRIPE, 16), jnp.float32),
        pltpu.SemaphoreType.DMA,
    ],
    compiler_params=pltpu.CompilerParams(use_tc_tiling_on_sc=False),
)
def _deg_kernel(srcd_hbm, dstd_hbm, ones_hbm, zeros_hbm, dop_hbm, dip_hbm,
                acc_o, acc_i, src_v, dst_v, ones_v, zv, sem):
    c = lax.axis_index("c")
    s = lax.axis_index("s")
    wid = s * NC + c
    base = s * STRIPE
    pltpu.sync_copy(zeros_hbm, zv)
    pltpu.sync_copy(zv, acc_o.at[pl.ds(base, STRIPE)])
    pltpu.sync_copy(zv, acc_i.at[pl.ds(base, STRIPE)])
    pltpu.sync_copy(ones_hbm, ones_v)
    plsc.subcore_barrier()
    pltpu.sync_copy(srcd_hbm.at[wid], src_v)
    pltpu.sync_copy(dstd_hbm.at[wid], dst_v)

    GD = 8

    def step(g, carry):
        for b in range(GD):
            j = g * GD + b
            pltpu.async_copy(ones_v, acc_o.at[src_v.at[j]], sem, add=True)
            pltpu.async_copy(ones_v, acc_i.at[dst_v.at[j]], sem, add=True)
        for b in range(2 * GD):
            pltpu.make_async_copy(ones_v, acc_o.at[src_v.at[0]], sem).wait()
        return carry

    lax.fori_loop(0, K // GD, step, 0)
    plsc.subcore_barrier()
    pltpu.sync_copy(acc_o.at[pl.ds(base, STRIPE)], zv)
    pltpu.sync_copy(zv, dop_hbm.at[c, pl.ds(base, STRIPE)])
    pltpu.sync_copy(acc_i.at[pl.ds(base, STRIPE)], zv)
    pltpu.sync_copy(zv, dip_hbm.at[c, pl.ds(base, STRIPE)])


NBUF = 4
LOOK = 2
W = NBUF


def _make_agg(F, CHF, n_phase=1):
    KF = EPW // CHF
    NWIN = KF // W

    _kern = functools.partial(
        pl.kernel,
        out_type=(jax.ShapeDtypeStruct((NC, NB, F), jnp.float32)
                  if n_phase == 1 else
                  tuple(jax.ShapeDtypeStruct((NC, NB, F), jnp.float32)
                        for _ in range(n_phase))),
        mesh=_mesh,
        scratch_types=[
            pltpu.VMEM_SHARED((NPAD, F), jnp.float32),
            pltpu.VMEM_SHARED((NB, F), jnp.float32),
            pltpu.VMEM((2, W, CHF), jnp.int32),
            pltpu.VMEM((2, W, CHF), jnp.int32),
        ] + [pltpu.VMEM((CHF, F), jnp.float32) for _ in range(NBUF)] + [
            pltpu.VMEM((ZR, F), jnp.float32),
        ] + [pltpu.SemaphoreType.DMA for _ in range(2 * NBUF + 2)],
        compiler_params=pltpu.CompilerParams(use_tc_tiling_on_sc=False),
    )

    def _agg_phase(u_hbm, srcg_hbm, dstd_hbm, p_hbm, accum, ucache,
                   srcw, dstw, rows, gsem, ssem, es, ed, wid, base, c, s,
                   zbuf):
        pltpu.sync_copy(u_hbm.at[pl.ds(base, STRIPE)],
                        ucache.at[pl.ds(base, STRIPE)])
        zv = jnp.zeros((16,), jnp.float32)
        for i in range(ZR):
            for f in range(F // 16):
                zbuf[i, pl.ds(f * 16, 16)] = zv
        for k in range(STRIPE // ZR):
            pltpu.sync_copy(zbuf, accum.at[pl.ds(base + k * ZR, ZR)])
        plsc.subcore_barrier()
        pltpu.sync_copy(srcg_hbm.at[wid, pl.ds(0, W)], srcw.at[0])
        pltpu.sync_copy(dstd_hbm.at[wid, pl.ds(0, W)], dstw.at[0])
        for b in range(LOOK):
            pltpu.async_copy(ucache.at[srcw.at[0, b]], rows[b], gsem[b])

        def win(w, carry):
            h = lax.rem(w, 2)
            oh = 1 - h

            for b in range(W):
                sl = (b + LOOK) % W
                j4 = w * W + b
                if b == 0:
                    @pl.when(w >= 1)
                    def _drain_prev_tail():
                        pltpu.make_async_copy(
                            rows[2], accum.at[dstw.at[h, 0]], ssem[2]).wait()
                        pltpu.make_async_copy(
                            rows[3], accum.at[dstw.at[h, 0]], ssem[3]).wait()

                    @pl.when(w + 1 < NWIN)
                    def _refill():
                        pltpu.async_copy(
                            srcg_hbm.at[wid, pl.ds((w + 1) * W, W)],
                            srcw.at[oh], es)
                        pltpu.async_copy(
                            dstd_hbm.at[wid, pl.ds((w + 1) * W, W)],
                            dstw.at[oh], ed)
                if b >= W - LOOK:
                    pltpu.make_async_copy(
                        rows[sl], accum.at[dstw.at[h, b]], ssem[sl]).wait()
                if b == W - LOOK:
                    @pl.when(w + 1 < NWIN)
                    def _wait_refill():
                        pltpu.make_async_copy(
                            srcg_hbm.at[wid, pl.ds(0, W)], srcw.at[oh], es).wait()
                        pltpu.make_async_copy(
                            dstd_hbm.at[wid, pl.ds(0, W)], dstw.at[oh], ed).wait()
                if b < W - LOOK:
                    pltpu.async_copy(
                        ucache.at[srcw.at[h, b + LOOK]], rows[sl], gsem[sl])
                else:
                    @pl.when(w + 1 < NWIN)
                    def _gather_next():
                        pltpu.async_copy(
                            ucache.at[srcw.at[oh, b + LOOK - W]], rows[sl],
                            gsem[sl])
                pltpu.make_async_copy(
                    ucache.at[srcw.at[h, b]], rows[b], gsem[b]).wait()
                pltpu.async_copy(rows[b], accum.at[dstw.at[h, b]], ssem[b],
                                 add=True)
            return carry

        lax.fori_loop(0, NWIN, win, 0)
        for b in range(W - LOOK, W):
            pltpu.make_async_copy(rows[b], accum.at[dstw.at[0, b]], ssem[b]).wait()
        plsc.subcore_barrier()
        for k in range(STRIPE // ZR):
            pltpu.sync_copy(accum.at[pl.ds(base + k * ZR, ZR)], zbuf)
            pltpu.sync_copy(zbuf, p_hbm.at[c, pl.ds(base + k * ZR, ZR)])

    if n_phase == 1:
        def agg(u0, srcg_hbm, dstd_hbm, p0,
                accum, ucache, srcw, dstw, r0, r1, r2, r3, zbuf,
                g0, g1, g2, g3, s0, s1, s2, s3, es, ed):
            c = lax.axis_index("c")
            s_ = lax.axis_index("s")
            wid = s_ * NC + c
            base = s_ * STRIPE
            _agg_phase(u0, srcg_hbm, dstd_hbm, p0, accum, ucache,
                       srcw, dstw, [r0, r1, r2, r3], [g0, g1, g2, g3],
                       [s0, s1, s2, s3], es, ed, wid, base, c, s_, zbuf)
    else:
        def agg(u0, u1, srcg_hbm, dstd_hbm, p0, p1,
                accum, ucache, srcw, dstw, r0, r1, r2, r3, zbuf,
                g0, g1, g2, g3, s0, s1, s2, s3, es, ed):
            c = lax.axis_index("c")
            s_ = lax.axis_index("s")
            wid = s_ * NC + c
            base = s_ * STRIPE
            for u_hbm, p_hbm in ((u0, p0), (u1, p1)):
                _agg_phase(u_hbm, srcg_hbm, dstd_hbm, p_hbm, accum, ucache,
                           srcw, dstw, [r0, r1, r2, r3], [g0, g1, g2, g3],
                           [s0, s1, s2, s3], es, ed, wid, base, c, s_, zbuf)

    return _kern(agg)


_agg64 = _make_agg(64, 128)
_agg64x2 = _make_agg(64, 128, n_phase=2)
_agg16 = _make_agg(16, 128)


R = 400
_GRID = N // R


def _row_spec(f):
    return pl.BlockSpec((R, f), lambda i: (i, 0))


def _part_spec(f):
    return pl.BlockSpec((NC, R, f), lambda i: (0, i, 0))


def _full_spec(a, b):
    return pl.BlockSpec((a, b), lambda i: (0, 0))


def _tc0_body(x_ref, dop_ref, dip_ref, u1a_ref, u1b_ref, ns_ref, nd_ref):
    ns8 = lax.rsqrt(jnp.maximum(dop_ref[0] + dop_ref[1], 1.0))
    nd8 = lax.rsqrt(jnp.maximum(dip_ref[0] + dip_ref[1], 1.0))
    ns_ref[...] = ns8
    nd_ref[...] = nd8
    xs = x_ref[...] * ns8[:, 0:1]
    u1a_ref[...] = xs[:, :64]
    u1b_ref[...] = xs[:, 64:]


_tc0 = pl.pallas_call(
    _tc0_body,
    grid=(_GRID,),
    in_specs=[_row_spec(128), _part_spec(16), _part_spec(16)],
    out_specs=(_row_spec(64), _row_spec(64), _row_spec(16), _row_spec(16)),
    out_shape=(jax.ShapeDtypeStruct((NB, 64), jnp.float32),
               jax.ShapeDtypeStruct((NB, 64), jnp.float32),
               jax.ShapeDtypeStruct((N, 16), jnp.float32),
               jax.ShapeDtypeStruct((N, 16), jnp.float32)),
)


def _tc1_body(pa_ref, pb_ref, ns_ref, nd_ref, w1a_ref, w1b_ref, b1_ref,
              w2_ref, u2_ref):
    nd1 = nd_ref[...][:, 0:1]
    ta = (pa_ref[0] + pa_ref[1]) * nd1
    tb = (pb_ref[0] + pb_ref[1]) * nd1
    h = (jnp.dot(ta, w1a_ref[...], preferred_element_type=jnp.float32)
         + jnp.dot(tb, w1b_ref[...], preferred_element_type=jnp.float32)
         + b1_ref[...])
    h = jnp.maximum(h, 0.0) * ns_ref[...][:, 0:1]
    u2_ref[...] = jnp.dot(h, w2_ref[...], preferred_element_type=jnp.float32)


_tc1 = pl.pallas_call(
    _tc1_body,
    grid=(_GRID,),
    in_specs=[_part_spec(64), _part_spec(64), _row_spec(16), _row_spec(16),
              _full_spec(64, 128), _full_spec(64, 128), _full_spec(1, 128),
              _full_spec(128, 64)],
    out_specs=_row_spec(64),
    out_shape=jax.ShapeDtypeStruct((NB, 64), jnp.float32),
)


def _tc2_body(p_ref, ns_ref, nd_ref, b2_ref, w3_ref, u3_ref):
    t = (p_ref[0] + p_ref[1]) * nd_ref[...][:, 0:1] + b2_ref[...]
    h = jnp.maximum(t, 0.0) * ns_ref[...][:, 0:1]
    u3_ref[...] = jnp.dot(h, w3_ref[...], preferred_element_type=jnp.float32)


_tc2 = pl.pallas_call(
    _tc2_body,
    grid=(_GRID,),
    in_specs=[_part_spec(64), _row_spec(16), _row_spec(16),
              _full_spec(1, 64), _full_spec(64, 16)],
    out_specs=_row_spec(16),
    out_shape=jax.ShapeDtypeStruct((NB, 16), jnp.float32),
)


def _tc3_body(p_ref, ns_ref, nd_ref, b3_ref, u4_ref):
    t = (p_ref[0] + p_ref[1]) * nd_ref[...][:, 0:1] + b3_ref[...]
    u4_ref[...] = jnp.maximum(t, 0.0) * ns_ref[...][:, 0:1]


_tc3 = pl.pallas_call(
    _tc3_body,
    grid=(_GRID,),
    in_specs=[_part_spec(16), _row_spec(16), _row_spec(16), _full_spec(1, 16)],
    out_specs=_row_spec(16),
    out_shape=jax.ShapeDtypeStruct((NB, 16), jnp.float32),
)


def _tc4_body(p_ref, nd_ref, w4_ref, b4_ref, out_ref):
    t = (p_ref[0] + p_ref[1]) * nd_ref[...][:, 0:1]
    out_ref[...] = jnp.dot(t, w4_ref[...],
                           preferred_element_type=jnp.float32) + b4_ref[...]


_tc4 = pl.pallas_call(
    _tc4_body,
    grid=(_GRID,),
    in_specs=[_part_spec(16), _row_spec(16), _full_spec(16, 40), _full_spec(1, 40)],
    out_specs=_row_spec(40),
    out_shape=jax.ShapeDtypeStruct((N, 40), jnp.float32),
)


def kernel(x, edge_index, W1, b1, W2, b2, W3, b3, W4, b4):
    ei = edge_index.astype(jnp.int32)
    src, dst = ei[0], ei[1]
    pad = E_PAD - E
    srcg = jnp.concatenate([src, jnp.zeros((pad,), jnp.int32)]).reshape(NW, K, CH)
    srcd = jnp.concatenate([src, jnp.full((pad,), NB, jnp.int32)]).reshape(NW, K, CH)
    dstd = jnp.concatenate([dst, jnp.full((pad,), NB, jnp.int32)]).reshape(NW, K, CH)
    ones8 = jnp.ones((CH, 16), jnp.float32)
    zeros8 = jnp.zeros((STRIPE, 16), jnp.float32)

    dop, dip = _deg_kernel(srcd, dstd, ones8, zeros8)
    u1a, u1b, ns8, nd8 = _tc0(x, dop, dip)
    p1a, p1b = _agg64x2(u1a, u1b, srcg, dstd)
    u2 = _tc1(p1a, p1b, ns8, nd8, W1[:64], W1[64:], b1.reshape(1, -1), W2)
    p2 = _agg64(u2, srcg, dstd)
    u3 = _tc2(p2, ns8, nd8, b2.reshape(1, -1), W3)
    p3 = _agg16(u3, srcg, dstd)
    u4 = _tc3(p3, ns8, nd8, b3.reshape(1, -1))
    p4 = _agg16(u4, srcg, dstd)
    return _tc4(p4, nd8, W4, b4.reshape(1, -1))


# speedup vs baseline: 1.1770x; 1.1770x over previous
import functools

import jax
import jax.numpy as jnp
from jax import lax
from jax.experimental import pallas as pl
from jax.experimental.pallas import tpu as pltpu
from jax.experimental.pallas import tpu_sc as plsc

N = 10000
E = 320000
NC, NS = 2, 16
NW = NC * NS
CH = 128
K = 80
EPW = K * CH
E_PAD = NW * EPW
NB = 10240
NPAD = NB + 8
STRIPE = NB // NS
ZR = 32

_mesh = plsc.VectorSubcoreMesh(core_axis_name="c", subcore_axis_name="s")


@functools.partial(
    pl.kernel,
    out_type=(jax.ShapeDtypeStruct((NC, NB, 16), jnp.float32),
              jax.ShapeDtypeStruct((NC, NB, 16), jnp.float32)),
    mesh=_mesh,
    scratch_types=[
        pltpu.VMEM_SHARED((NPAD, 16), jnp.float32),
        pltpu.VMEM_SHARED((NPAD, 16), jnp.float32),
        pltpu.VMEM((K, CH), jnp.int32),
        pltpu.VMEM((K, CH), jnp.int32),
        pltpu.VMEM((CH, 16), jnp.float32),
        pltpu.VMEM((STRIPE, 16), jnp.float32),
        pltpu.SemaphoreType.DMA,
    ],
    compiler_params=pltpu.CompilerParams(use_tc_tiling_on_sc=False),
)
def _deg_kernel(srcd_hbm, dstd_hbm, ones_hbm, zeros_hbm, dop_hbm, dip_hbm,
                acc_o, acc_i, src_v, dst_v, ones_v, zv, sem):
    c = lax.axis_index("c")
    s = lax.axis_index("s")
    wid = s * NC + c
    base = s * STRIPE
    pltpu.sync_copy(zeros_hbm, zv)
    pltpu.sync_copy(zv, acc_o.at[pl.ds(base, STRIPE)])
    pltpu.sync_copy(zv, acc_i.at[pl.ds(base, STRIPE)])
    pltpu.sync_copy(ones_hbm, ones_v)
    plsc.subcore_barrier()
    pltpu.sync_copy(srcd_hbm.at[wid], src_v)
    pltpu.sync_copy(dstd_hbm.at[wid], dst_v)

    GD = 8

    def step(g, carry):
        for b in range(GD):
            j = g * GD + b
            pltpu.async_copy(ones_v, acc_o.at[src_v.at[j]], sem, add=True)
            pltpu.async_copy(ones_v, acc_i.at[dst_v.at[j]], sem, add=True)
        for b in range(2 * GD):
            pltpu.make_async_copy(ones_v, acc_o.at[src_v.at[0]], sem).wait()
        return carry

    lax.fori_loop(0, K // GD, step, 0)
    plsc.subcore_barrier()
    pltpu.sync_copy(acc_o.at[pl.ds(base, STRIPE)], zv)
    pltpu.sync_copy(zv, dop_hbm.at[c, pl.ds(base, STRIPE)])
    pltpu.sync_copy(acc_i.at[pl.ds(base, STRIPE)], zv)
    pltpu.sync_copy(zv, dip_hbm.at[c, pl.ds(base, STRIPE)])


NBUF = 4
LOOK = 2
W = NBUF


def _make_agg(F, CHF, n_phase=1, dt=jnp.float32):
    KF = EPW // CHF
    NWIN = KF // W

    _kern = functools.partial(
        pl.kernel,
        out_type=(jax.ShapeDtypeStruct((NC, NB, F), dt)
                  if n_phase == 1 else
                  tuple(jax.ShapeDtypeStruct((NC, NB, F), dt)
                        for _ in range(n_phase))),
        mesh=_mesh,
        scratch_types=[
            pltpu.VMEM_SHARED((NPAD, F), dt),
            pltpu.VMEM_SHARED((NB, F), dt),
            pltpu.VMEM((2, W, CHF), jnp.int32),
            pltpu.VMEM((2, W, CHF), jnp.int32),
        ] + [pltpu.VMEM((CHF, F), dt) for _ in range(NBUF)] + [
            pltpu.VMEM((ZR, F), dt),
        ] + [pltpu.SemaphoreType.DMA for _ in range(2 * NBUF + 2)],
        compiler_params=pltpu.CompilerParams(use_tc_tiling_on_sc=False),
    )

    def _agg_phase(u_hbm, srcg_hbm, dstd_hbm, p_hbm, accum, ucache,
                   srcw, dstw, rows, gsem, ssem, es, ed, wid, base, c, s,
                   zbuf):
        pltpu.sync_copy(u_hbm.at[pl.ds(base, STRIPE)],
                        ucache.at[pl.ds(base, STRIPE)])
        ln = 16 if dt == jnp.float32 else 32
        zv = jnp.zeros((ln,), dt)
        for i in range(ZR):
            for f in range(F // ln):
                zbuf[i, pl.ds(f * ln, ln)] = zv
        for k in range(STRIPE // ZR):
            pltpu.sync_copy(zbuf, accum.at[pl.ds(base + k * ZR, ZR)])
        plsc.subcore_barrier()
        pltpu.sync_copy(srcg_hbm.at[wid, pl.ds(0, W)], srcw.at[0])
        pltpu.sync_copy(dstd_hbm.at[wid, pl.ds(0, W)], dstw.at[0])
        for b in range(LOOK):
            pltpu.async_copy(ucache.at[srcw.at[0, b]], rows[b], gsem[b])

        def win(w, carry):
            h = lax.rem(w, 2)
            oh = 1 - h

            for b in range(W):
                sl = (b + LOOK) % W
                j4 = w * W + b
                if b == 0:
                    @pl.when(w >= 1)
                    def _drain_prev_tail():
                        pltpu.make_async_copy(
                            rows[2], accum.at[dstw.at[h, 0]], ssem[2]).wait()
                        pltpu.make_async_copy(
                            rows[3], accum.at[dstw.at[h, 0]], ssem[3]).wait()

                    @pl.when(w + 1 < NWIN)
                    def _refill():
                        pltpu.async_copy(
                            srcg_hbm.at[wid, pl.ds((w + 1) * W, W)],
                            srcw.at[oh], es)
                        pltpu.async_copy(
                            dstd_hbm.at[wid, pl.ds((w + 1) * W, W)],
                            dstw.at[oh], ed)
                if b >= W - LOOK:
                    pltpu.make_async_copy(
                        rows[sl], accum.at[dstw.at[h, b]], ssem[sl]).wait()
                if b == W - LOOK:
                    @pl.when(w + 1 < NWIN)
                    def _wait_refill():
                        pltpu.make_async_copy(
                            srcg_hbm.at[wid, pl.ds(0, W)], srcw.at[oh], es).wait()
                        pltpu.make_async_copy(
                            dstd_hbm.at[wid, pl.ds(0, W)], dstw.at[oh], ed).wait()
                if b < W - LOOK:
                    pltpu.async_copy(
                        ucache.at[srcw.at[h, b + LOOK]], rows[sl], gsem[sl])
                else:
                    @pl.when(w + 1 < NWIN)
                    def _gather_next():
                        pltpu.async_copy(
                            ucache.at[srcw.at[oh, b + LOOK - W]], rows[sl],
                            gsem[sl])
                pltpu.make_async_copy(
                    ucache.at[srcw.at[h, b]], rows[b], gsem[b]).wait()
                pltpu.async_copy(rows[b], accum.at[dstw.at[h, b]], ssem[b],
                                 add=True)
            return carry

        lax.fori_loop(0, NWIN, win, 0)
        for b in range(W - LOOK, W):
            pltpu.make_async_copy(rows[b], accum.at[dstw.at[0, b]], ssem[b]).wait()
        plsc.subcore_barrier()
        for k in range(STRIPE // ZR):
            pltpu.sync_copy(accum.at[pl.ds(base + k * ZR, ZR)], zbuf)
            pltpu.sync_copy(zbuf, p_hbm.at[c, pl.ds(base + k * ZR, ZR)])

    if n_phase == 1:
        def agg(u0, srcg_hbm, dstd_hbm, p0,
                accum, ucache, srcw, dstw, r0, r1, r2, r3, zbuf,
                g0, g1, g2, g3, s0, s1, s2, s3, es, ed):
            c = lax.axis_index("c")
            s_ = lax.axis_index("s")
            wid = s_ * NC + c
            base = s_ * STRIPE
            _agg_phase(u0, srcg_hbm, dstd_hbm, p0, accum, ucache,
                       srcw, dstw, [r0, r1, r2, r3], [g0, g1, g2, g3],
                       [s0, s1, s2, s3], es, ed, wid, base, c, s_, zbuf)
    else:
        def agg(u0, u1, srcg_hbm, dstd_hbm, p0, p1,
                accum, ucache, srcw, dstw, r0, r1, r2, r3, zbuf,
                g0, g1, g2, g3, s0, s1, s2, s3, es, ed):
            c = lax.axis_index("c")
            s_ = lax.axis_index("s")
            wid = s_ * NC + c
            base = s_ * STRIPE
            for u_hbm, p_hbm in ((u0, p0), (u1, p1)):
                _agg_phase(u_hbm, srcg_hbm, dstd_hbm, p_hbm, accum, ucache,
                           srcw, dstw, [r0, r1, r2, r3], [g0, g1, g2, g3],
                           [s0, s1, s2, s3], es, ed, wid, base, c, s_, zbuf)

    return _kern(agg)


_agg64 = _make_agg(64, 128, dt=jnp.bfloat16)
_agg64x2 = _make_agg(64, 128, n_phase=2, dt=jnp.bfloat16)
_agg16 = _make_agg(16, 128)


R = 400
_GRID = N // R


def _row_spec(f):
    return pl.BlockSpec((R, f), lambda i: (i, 0))


def _part_spec(f):
    return pl.BlockSpec((NC, R, f), lambda i: (0, i, 0))


def _full_spec(a, b):
    return pl.BlockSpec((a, b), lambda i: (0, 0))


def _tc0_body(x_ref, dop_ref, dip_ref, u1a_ref, u1b_ref, ns_ref, nd_ref):
    ns8 = lax.rsqrt(jnp.maximum(dop_ref[0] + dop_ref[1], 1.0))
    nd8 = lax.rsqrt(jnp.maximum(dip_ref[0] + dip_ref[1], 1.0))
    ns_ref[...] = ns8
    nd_ref[...] = nd8
    xs = x_ref[...] * ns8[:, 0:1]
    u1a_ref[...] = xs[:, :64].astype(jnp.bfloat16)
    u1b_ref[...] = xs[:, 64:].astype(jnp.bfloat16)


_tc0 = pl.pallas_call(
    _tc0_body,
    grid=(_GRID,),
    in_specs=[_row_spec(128), _part_spec(16), _part_spec(16)],
    out_specs=(_row_spec(64), _row_spec(64), _row_spec(16), _row_spec(16)),
    out_shape=(jax.ShapeDtypeStruct((NB, 64), jnp.bfloat16),
               jax.ShapeDtypeStruct((NB, 64), jnp.bfloat16),
               jax.ShapeDtypeStruct((N, 16), jnp.float32),
               jax.ShapeDtypeStruct((N, 16), jnp.float32)),
)


def _tc1_body(pa_ref, pb_ref, ns_ref, nd_ref, w1a_ref, w1b_ref, b1_ref,
              w2_ref, u2_ref):
    nd1 = nd_ref[...][:, 0:1]
    ta = (pa_ref[0].astype(jnp.float32) + pa_ref[1].astype(jnp.float32)) * nd1
    tb = (pb_ref[0].astype(jnp.float32) + pb_ref[1].astype(jnp.float32)) * nd1
    h = (jnp.dot(ta, w1a_ref[...], preferred_element_type=jnp.float32)
         + jnp.dot(tb, w1b_ref[...], preferred_element_type=jnp.float32)
         + b1_ref[...])
    h = jnp.maximum(h, 0.0) * ns_ref[...][:, 0:1]
    u2_ref[...] = jnp.dot(h, w2_ref[...],
                          preferred_element_type=jnp.float32).astype(jnp.bfloat16)


_tc1 = pl.pallas_call(
    _tc1_body,
    grid=(_GRID,),
    in_specs=[_part_spec(64), _part_spec(64), _row_spec(16), _row_spec(16),
              _full_spec(64, 128), _full_spec(64, 128), _full_spec(1, 128),
              _full_spec(128, 64)],
    out_specs=_row_spec(64),
    out_shape=jax.ShapeDtypeStruct((NB, 64), jnp.bfloat16),
)


def _tc2_body(p_ref, ns_ref, nd_ref, b2_ref, w3_ref, u3_ref):
    t = ((p_ref[0].astype(jnp.float32) + p_ref[1].astype(jnp.float32))
         * nd_ref[...][:, 0:1] + b2_ref[...])
    h = jnp.maximum(t, 0.0) * ns_ref[...][:, 0:1]
    u3_ref[...] = jnp.dot(h, w3_ref[...], preferred_element_type=jnp.float32)


_tc2 = pl.pallas_call(
    _tc2_body,
    grid=(_GRID,),
    in_specs=[_part_spec(64), _row_spec(16), _row_spec(16),
              _full_spec(1, 64), _full_spec(64, 16)],
    out_specs=_row_spec(16),
    out_shape=jax.ShapeDtypeStruct((NB, 16), jnp.float32),
)


def _tc3_body(p_ref, ns_ref, nd_ref, b3_ref, u4_ref):
    t = (p_ref[0] + p_ref[1]) * nd_ref[...][:, 0:1] + b3_ref[...]
    u4_ref[...] = jnp.maximum(t, 0.0) * ns_ref[...][:, 0:1]


_tc3 = pl.pallas_call(
    _tc3_body,
    grid=(_GRID,),
    in_specs=[_part_spec(16), _row_spec(16), _row_spec(16), _full_spec(1, 16)],
    out_specs=_row_spec(16),
    out_shape=jax.ShapeDtypeStruct((NB, 16), jnp.float32),
)


def _tc4_body(p_ref, nd_ref, w4_ref, b4_ref, out_ref):
    t = (p_ref[0] + p_ref[1]) * nd_ref[...][:, 0:1]
    out_ref[...] = jnp.dot(t, w4_ref[...],
                           preferred_element_type=jnp.float32) + b4_ref[...]


_tc4 = pl.pallas_call(
    _tc4_body,
    grid=(_GRID,),
    in_specs=[_part_spec(16), _row_spec(16), _full_spec(16, 40), _full_spec(1, 40)],
    out_specs=_row_spec(40),
    out_shape=jax.ShapeDtypeStruct((N, 40), jnp.float32),
)


def kernel(x, edge_index, W1, b1, W2, b2, W3, b3, W4, b4):
    ei = edge_index.astype(jnp.int32)
    src, dst = ei[0], ei[1]
    pad = E_PAD - E
    srcg = jnp.concatenate([src, jnp.zeros((pad,), jnp.int32)]).reshape(NW, K, CH)
    srcd = jnp.concatenate([src, jnp.full((pad,), NB, jnp.int32)]).reshape(NW, K, CH)
    dstd = jnp.concatenate([dst, jnp.full((pad,), NB, jnp.int32)]).reshape(NW, K, CH)
    ones8 = jnp.ones((CH, 16), jnp.float32)
    zeros8 = jnp.zeros((STRIPE, 16), jnp.float32)

    dop, dip = _deg_kernel(srcd, dstd, ones8, zeros8)
    u1a, u1b, ns8, nd8 = _tc0(x, dop, dip)
    p1a, p1b = _agg64x2(u1a, u1b, srcg, dstd)
    u2 = _tc1(p1a, p1b, ns8, nd8, W1[:64], W1[64:], b1.reshape(1, -1), W2)
    p2 = _agg64(u2, srcg, dstd)
    u3 = _tc2(p2, ns8, nd8, b2.reshape(1, -1), W3)
    p3 = _agg16(u3, srcg, dstd)
    u4 = _tc3(p3, ns8, nd8, b3.reshape(1, -1))
    p4 = _agg16(u4, srcg, dstd)
    return _tc4(p4, nd8, W4, b4.reshape(1, -1))
